# baseline (speedup 1.0000x reference)
import functools

import jax
import jax.numpy as jnp
from jax import lax
from jax.experimental import pallas as pl
from jax.experimental.pallas import tpu as pltpu
from jax.experimental.pallas import tpu_sc as plsc

_SMOOTHING = 0.1
_CONFIDENCE = 1.0 - _SMOOTHING
_V = 100000
_B = 1024
_FILL = _SMOOTHING / (_V - 2)

_TOTAL = _B * _V
_NTC = 2
_NCHUNK = 100
_CHUNK = _TOTAL // _NCHUNK
_PER_TC = _NCHUNK // _NTC
_INIT_SLICE = 8192
_L = 16
_NC, _NS = 2, 16
_NW = _NC * _NS
_PER_W = _B // _NW


@functools.partial(
    pl.kernel,
    mesh=pltpu.create_tensorcore_mesh("tc", num_cores=_NTC),
    scratch_types=[
        pltpu.VMEM((_CHUNK,), jnp.float32),
        pltpu.SemaphoreType.DMA,
    ],
)
def _tc_fill(out_hbm, scratch, sem):
    cid = lax.axis_index("tc")

    def init(i, carry):
        scratch[pl.ds(i * _INIT_SLICE, _INIT_SLICE)] = jnp.full(
            (_INIT_SLICE,), _FILL, jnp.float32)
        return carry

    lax.fori_loop(0, _CHUNK // _INIT_SLICE, init, 0)
    base = cid * (_PER_TC * _CHUNK)
    for j in range(_PER_TC):
        pltpu.make_async_copy(
            scratch, out_hbm.at[pl.ds(base + j * _CHUNK, _CHUNK)], sem
        ).start()
    for j in range(_PER_TC):
        pltpu.make_async_copy(
            scratch, out_hbm.at[pl.ds(base, _CHUNK)], sem).wait()


@functools.partial(
    pl.kernel,
    mesh=plsc.VectorSubcoreMesh(core_axis_name="c", subcore_axis_name="s"),
    scratch_types=[
        pltpu.VMEM((_PER_W,), jnp.int32),
        pltpu.VMEM((2 * _PER_W,), jnp.int32),
        pltpu.VMEM((2 * _PER_W,), jnp.float32),
        pltpu.SemaphoreType.DMA,
    ],
)
def _sc_scatter(ids_hbm, out_hbm, ids_v, idx_v, val_v, sem):
    wid = lax.axis_index("s") * _NC + lax.axis_index("c")
    base = wid * _PER_W
    pltpu.async_copy(ids_hbm.at[pl.ds(base, _PER_W)], ids_v, sem).wait()
    zero = jnp.zeros((_L,), jnp.float32)
    conf = jnp.full((_L,), _CONFIDENCE, jnp.float32)
    for g in range(_PER_W // _L):
        ids = ids_v[pl.ds(g * _L, _L)]
        rows = lax.iota(jnp.int32, _L) + (base + g * _L)
        idx_v[pl.ds(g * _L, _L)] = rows * _V + ids
        idx_v[pl.ds(_PER_W + g * _L, _L)] = rows * _V
        val_v[pl.ds(g * _L, _L)] = jnp.where(ids == 0, zero, conf)
        val_v[pl.ds(_PER_W + g * _L, _L)] = zero
    pltpu.async_copy(val_v, out_hbm.at[idx_v], sem).wait()


@jax.jit
def _run(ids):
    buf = jax.new_ref(lax.empty((_TOTAL,), jnp.float32))
    _tc_fill(buf)
    _sc_scatter(ids, buf)
    return jax.freeze(buf).reshape(_B, _V)


def kernel(tgt_tok_id):
    return _run(tgt_tok_id.reshape(-1).astype(jnp.int32))


# speedup vs baseline: 8.1263x; 8.1263x over previous
import jax
import jax.numpy as jnp
from jax.experimental import pallas as pl

_SMOOTHING = 0.1
_CONFIDENCE = 1.0 - _SMOOTHING
_V = 100000
_B = 1024
_FILL = _SMOOTHING / (_V - 2)

_VR = 2000


def _smooth_block(ids_ref, out_ref):
    j = pl.program_id(0)
    ids = ids_ref[0, :]
    vocab = jax.lax.broadcasted_iota(jnp.int32, (_VR, _B), 0) + j * _VR
    val = jnp.where(vocab == ids[None, :], _CONFIDENCE, _FILL)
    out_ref[...] = jnp.where(vocab == 0, 0.0, val)


def kernel(tgt_tok_id):
    ids = tgt_tok_id.reshape(1, _B).astype(jnp.int32)
    out_t = pl.pallas_call(
        _smooth_block,
        grid=(_V // _VR,),
        in_specs=[pl.BlockSpec((1, _B), lambda j: (0, 0))],
        out_specs=pl.BlockSpec((_VR, _B), lambda j: (j, 0)),
        out_shape=jax.ShapeDtypeStruct((_V, _B), jnp.float32),
    )(ids)
    return jnp.transpose(out_t)
